# Initial kernel scaffold; baseline (speedup 1.0000x reference)
import functools

import jax
import jax.numpy as jnp
from jax import lax
from jax.experimental import pallas as pl
from jax.experimental.pallas import tpu as pltpu
from jax.experimental.pallas import tpu_sc as plsc

NC = 2
NS = 16
L = 16
CH = 128
DW = 16
G = 64


def _zero_fill(buf, rows, width):
  zero = jnp.zeros((L,), jnp.float32)

  def body(r, _):
    for l in range(width // L):
      buf[r, pl.ds(l * L, L)] = zero
    return 0

  lax.fori_loop(0, rows, body, 0)


def _make_sc_degree(N_pad, E_pad):
  e_per_tile = E_pad // (NC * NS)
  n_ch = e_per_tile // CH
  rows_per_tile = N_pad // NS
  ZR = 128
  mesh = plsc.VectorSubcoreMesh(core_axis_name="c", subcore_axis_name="s")

  @functools.partial(
      pl.kernel,
      out_type=jax.ShapeDtypeStruct((NC, N_pad, DW), jnp.float32),
      mesh=mesh,
      scratch_types=[
          pltpu.VMEM_SHARED((N_pad, DW), jnp.float32),
          pltpu.VMEM((1, CH), jnp.int32),
          pltpu.VMEM((CH, DW), jnp.float32),
          pltpu.VMEM((ZR, DW), jnp.float32),
      ],
  )
  def deg_kernel(dst_hbm, out_hbm, acc, idxb, ones_v, zbuf):
    c = lax.axis_index("c")
    s = lax.axis_index("s")

    _zero_fill(zbuf, ZR, DW)
    one = jnp.full((L,), 1.0, jnp.float32)

    def fill_ones(r, _):
      ones_v[r, pl.ds(0, L)] = one
      return 0

    lax.fori_loop(0, CH, fill_ones, 0)

    row0 = s * rows_per_tile
    for k in range(rows_per_tile // ZR):
      pltpu.sync_copy(zbuf, acc.at[pl.ds(row0 + k * ZR, ZR)])
    plsc.subcore_barrier()

    base0 = (c * NS + s) * e_per_tile

    def body(j, _):
      b = pl.multiple_of(base0 + j * CH, CH)
      pltpu.sync_copy(dst_hbm.at[pl.ds(b, CH)], idxb.at[0])
      pltpu.sync_copy(ones_v, acc.at[idxb.at[0]], add=True)
      return 0

    lax.fori_loop(0, n_ch, body, 0)
    plsc.subcore_barrier()

    for k in range(rows_per_tile // ZR):
      r = row0 + k * ZR
      pltpu.sync_copy(acc.at[pl.ds(r, ZR)], out_hbm.at[c, pl.ds(r, ZR)])

  return deg_kernel


def _make_sc_agg(N_pad, E_pad, D):
  e_per_tile = E_pad // (NC * NS)
  n_ch = e_per_tile // CH
  rows_per_tile = N_pad // NS
  ZR = 128
  mesh = plsc.VectorSubcoreMesh(core_axis_name="c", subcore_axis_name="s")

  @functools.partial(
      pl.kernel,
      out_type=jax.ShapeDtypeStruct((NC, N_pad, D), jnp.float32),
      mesh=mesh,
      scratch_types=[
          pltpu.VMEM_SHARED((N_pad, D), jnp.float32),
          pltpu.VMEM((2, CH), jnp.int32),
          pltpu.VMEM((2, CH), jnp.int32),
          pltpu.VMEM((2, CH, D), jnp.float32),
          pltpu.VMEM((ZR, D), jnp.float32),
          pltpu.SemaphoreType.DMA,
          pltpu.SemaphoreType.DMA,
      ],
  )
  def agg_kernel(h_hbm, src_hbm, dst_hbm, out_hbm, acc, srcb, dstb, rowb,
                 zbuf, gsem0, gsem1):
    c = lax.axis_index("c")
    s = lax.axis_index("s")

    _zero_fill(zbuf, ZR, D)
    row0 = s * rows_per_tile
    for k in range(rows_per_tile // ZR):
      pltpu.sync_copy(zbuf, acc.at[pl.ds(row0 + k * ZR, ZR)])
    plsc.subcore_barrier()

    base0 = (c * NS + s) * e_per_tile

    def load_idx(j, slot):
      b = pl.multiple_of(base0 + j * CH, CH)
      pltpu.sync_copy(src_hbm.at[pl.ds(b, CH)], srcb.at[slot])
      pltpu.sync_copy(dst_hbm.at[pl.ds(b, CH)], dstb.at[slot])

    load_idx(0, 0)
    pltpu.async_copy(h_hbm.at[srcb.at[0]], rowb.at[0], gsem0)

    def body(j, _):
      slot = lax.rem(j, 2)

      @pl.when(slot == 0)
      def _():
        pltpu.make_async_copy(h_hbm.at[srcb.at[0]], rowb.at[0], gsem0).wait()

        @pl.when(j + 1 < n_ch)
        def _():
          load_idx(j + 1, 1)
          pltpu.async_copy(h_hbm.at[srcb.at[1]], rowb.at[1], gsem1)
        pltpu.sync_copy(rowb.at[0], acc.at[dstb.at[0]], add=True)

      @pl.when(slot == 1)
      def _():
        pltpu.make_async_copy(h_hbm.at[srcb.at[1]], rowb.at[1], gsem1).wait()

        @pl.when(j + 1 < n_ch)
        def _():
          load_idx(j + 1, 0)
          pltpu.async_copy(h_hbm.at[srcb.at[0]], rowb.at[0], gsem0)
        pltpu.sync_copy(rowb.at[1], acc.at[dstb.at[1]], add=True)

      return 0

    lax.fori_loop(0, n_ch, body, 0)
    plsc.subcore_barrier()

    for k in range(rows_per_tile // ZR):
      r = row0 + k * ZR
      pltpu.sync_copy(acc.at[pl.ds(r, ZR)], out_hbm.at[c, pl.ds(r, ZR)])

  return agg_kernel


def _make_tc_first(N, N_pad, D):
  def body(deg_ref, x_ref, w_ref, dinv_ref, h_ref):
    deg = deg_ref[0, :, 0:1] + deg_ref[1, :, 0:1] + 1.0
    rid = lax.broadcasted_iota(jnp.int32, (N_pad, 1), 0)
    dinv = jnp.where(rid < N, lax.rsqrt(deg), 0.0)
    dinv_ref[...] = dinv
    h = jnp.dot(x_ref[...], w_ref[...], preferred_element_type=jnp.float32)
    h_ref[...] = h * dinv

  return pl.pallas_call(
      body,
      out_shape=[
          jax.ShapeDtypeStruct((N_pad, 1), jnp.float32),
          jax.ShapeDtypeStruct((N_pad, D), jnp.float32),
      ],
  )


def _make_tc_layer(N, N_pad, D, residual, has_next):
  def body(*refs):
    if residual:
      (parts_ref, hprev_ref, dinv_ref, b_ref, g_ref, be_ref, xprev_ref,
       *rest) = refs
    else:
      parts_ref, hprev_ref, dinv_ref, b_ref, g_ref, be_ref, *rest = refs
    if has_next:
      wn_ref, x_out, h_out = rest
    else:
      (x_out,) = rest

    dinv = dinv_ref[...]
    agg = parts_ref[0] + parts_ref[1] + hprev_ref[...]
    conv = agg * dinv + b_ref[...]
    rid = lax.broadcasted_iota(jnp.int32, (N_pad, 1), 0)
    mask = rid < N
    r = jnp.where(mask, jax.nn.relu(conv), 0.0)
    mu = jnp.sum(r, axis=0, keepdims=True) * (1.0 / N)
    ctr = jnp.where(mask, r - mu, 0.0)
    var = jnp.sum(ctr * ctr, axis=0, keepdims=True) * (1.0 / N)
    xn = (r - mu) * lax.rsqrt(var + 1e-5) * g_ref[...] + be_ref[...]
    if residual:
      xn = xn + xprev_ref[...]
    x_out[...] = xn
    if has_next:
      h = jnp.dot(xn, wn_ref[...], preferred_element_type=jnp.float32)
      h_out[...] = h * dinv

  out_shape = [jax.ShapeDtypeStruct((N_pad, D), jnp.float32)]
  if has_next:
    out_shape.append(jax.ShapeDtypeStruct((N_pad, D), jnp.float32))
  return pl.pallas_call(body, out_shape=out_shape)


def _make_tc_final(N_pad, D):
  def body(x_ref, batch_ref, wr_ref, br_ref, pred_ref, mean_ref):
    bvals = batch_ref[...]
    gid = lax.broadcasted_iota(jnp.int32, (N_pad, G), 1)
    m = (bvals == gid).astype(jnp.float32)
    dn = (((0,), (0,)), ((), ()))
    sums = lax.dot_general(m, x_ref[...], dn,
                           preferred_element_type=jnp.float32)
    ones = jnp.ones((N_pad, 1), jnp.float32)
    counts = lax.dot_general(m, ones, dn,
                             preferred_element_type=jnp.float32)
    mean = sums / jnp.maximum(counts, 1.0)
    mean_ref[...] = mean
    pred_ref[...] = jnp.dot(mean, wr_ref[...],
                            preferred_element_type=jnp.float32) + br_ref[...]

  return pl.pallas_call(
      body,
      out_shape=[
          jax.ShapeDtypeStruct((G, 1), jnp.float32),
          jax.ShapeDtypeStruct((G, D), jnp.float32),
      ],
  )


def kernel(X, edge_index, batch, W1, b1, g1, be1, W2, b2, g2, be2, W3, b3, g3,
           be3, W4, b4, g4, be4, W5, b5, g5, be5, Wr, br):
  N, D = X.shape
  E = edge_index.shape[1]

  rows_unit = NS * 128
  N_pad = ((N + 1 + rows_unit - 1) // rows_unit) * rows_unit
  e_unit = NC * NS * CH
  E_pad = ((E + e_unit - 1) // e_unit) * e_unit

  Xp = jnp.pad(X, ((0, N_pad - N), (0, 0)))
  src = jnp.pad(edge_index[0], (0, E_pad - E), constant_values=N)
  dst = jnp.pad(edge_index[1], (0, E_pad - E), constant_values=N)
  batch2 = jnp.pad(batch, (0, N_pad - N), constant_values=G).reshape(N_pad, 1)

  sc_degree = _make_sc_degree(N_pad, E_pad)
  sc_agg = _make_sc_agg(N_pad, E_pad, D)
  tc_first = _make_tc_first(N, N_pad, D)
  tc_final = _make_tc_final(N_pad, D)

  deg_parts = sc_degree(dst)
  dinv, hp = tc_first(deg_parts, Xp, W1)

  layer_params = [
      (b1, g1, be1, W2),
      (b2, g2, be2, W3),
      (b3, g3, be3, W4),
      (b4, g4, be4, W5),
      (b5, g5, be5, None),
  ]
  x_cur = None
  for i, (b, g, be, wn) in enumerate(layer_params):
    parts = sc_agg(hp, src, dst)
    residual = i > 0
    has_next = wn is not None
    tc_layer = _make_tc_layer(N, N_pad, D, residual, has_next)
    args = [parts, hp, dinv, b.reshape(1, D), g.reshape(1, D),
            be.reshape(1, D)]
    if residual:
      args.append(x_cur)
    if has_next:
      args.append(wn)
      x_cur, hp = tc_layer(*args)
    else:
      (x_cur,) = tc_layer(*args)

  pred, mean = tc_final(x_cur, batch2, Wr, br.reshape(1, 1))
  return (pred, mean)


# speedup vs baseline: 7.3750x; 7.3750x over previous
import functools

import jax
import jax.numpy as jnp
from jax import lax
from jax.experimental import pallas as pl
from jax.experimental.pallas import tpu as pltpu
from jax.experimental.pallas import tpu_sc as plsc

NC = 2
NS = 16
L = 16
CH = 128
G = 64


def _zero_fill(buf, rows, width):
  zero = jnp.zeros((L,), jnp.float32)

  def body(r, _):
    for l in range(width // L):
      buf[r, pl.ds(l * L, L)] = zero
    return 0

  lax.fori_loop(0, rows, body, 0)


def _make_sc_agg(N_pad, E_pad, D):
  e_per_tile = E_pad // (NC * NS)
  n_ch = e_per_tile // CH
  rows_per_tile = N_pad // NS
  ZR = 128
  mesh = plsc.VectorSubcoreMesh(core_axis_name="c", subcore_axis_name="s")

  @functools.partial(
      pl.kernel,
      out_type=jax.ShapeDtypeStruct((NC, N_pad, D), jnp.float32),
      mesh=mesh,
      scratch_types=[
          pltpu.VMEM_SHARED((N_pad, D), jnp.float32),
          pltpu.VMEM((2, CH), jnp.int32),
          pltpu.VMEM((2, CH), jnp.int32),
          pltpu.VMEM((2, CH, D), jnp.float32),
          pltpu.SemaphoreType.DMA,
          pltpu.SemaphoreType.DMA,
      ],
  )
  def agg_kernel(h_hbm, src_hbm, dst_hbm, out_hbm, acc, srcb, dstb, rowb,
                 gsem0, gsem1):
    c = lax.axis_index("c")
    s = lax.axis_index("s")

    _zero_fill(rowb.at[0], ZR, D)
    row0 = s * rows_per_tile
    for k in range(rows_per_tile // ZR):
      pltpu.sync_copy(rowb.at[0], acc.at[pl.ds(row0 + k * ZR, ZR)])
    plsc.subcore_barrier()

    base0 = (c * NS + s) * e_per_tile

    def load_idx(j, slot):
      b = pl.multiple_of(base0 + j * CH, CH)
      pltpu.sync_copy(src_hbm.at[pl.ds(b, CH)], srcb.at[slot])
      pltpu.sync_copy(dst_hbm.at[pl.ds(b, CH)], dstb.at[slot])

    load_idx(0, 0)
    pltpu.async_copy(h_hbm.at[srcb.at[0]], rowb.at[0], gsem0)

    def body(j, _):
      slot = lax.rem(j, 2)

      @pl.when(slot == 0)
      def _():
        pltpu.make_async_copy(h_hbm.at[srcb.at[0]], rowb.at[0], gsem0).wait()

        @pl.when(j + 1 < n_ch)
        def _():
          load_idx(j + 1, 1)
          pltpu.async_copy(h_hbm.at[srcb.at[1]], rowb.at[1], gsem1)
        pltpu.sync_copy(rowb.at[0], acc.at[dstb.at[0]], add=True)

      @pl.when(slot == 1)
      def _():
        pltpu.make_async_copy(h_hbm.at[srcb.at[1]], rowb.at[1], gsem1).wait()

        @pl.when(j + 1 < n_ch)
        def _():
          load_idx(j + 1, 0)
          pltpu.async_copy(h_hbm.at[srcb.at[0]], rowb.at[0], gsem0)
        pltpu.sync_copy(rowb.at[1], acc.at[dstb.at[1]], add=True)

      return 0

    lax.fori_loop(0, n_ch, body, 0)
    plsc.subcore_barrier()

    for k in range(rows_per_tile // ZR):
      r = row0 + k * ZR
      pltpu.sync_copy(acc.at[pl.ds(r, ZR)], out_hbm.at[c, pl.ds(r, ZR)])

  return agg_kernel


def _make_tc_first(N, N_pad, D):
  def body(deg_ref, x_ref, w_ref, dinv_ref, h_ref):
    deg = deg_ref[0, :, 0:1] + deg_ref[1, :, 0:1] + 1.0
    rid = lax.broadcasted_iota(jnp.int32, (N_pad, 1), 0)
    dinv = jnp.where(rid < N, lax.rsqrt(deg), 0.0)
    dinv_ref[...] = dinv
    h = jnp.dot(x_ref[...], w_ref[...], preferred_element_type=jnp.float32)
    h_ref[...] = h * dinv

  return pl.pallas_call(
      body,
      out_shape=[
          jax.ShapeDtypeStruct((N_pad, 1), jnp.float32),
          jax.ShapeDtypeStruct((N_pad, D), jnp.float32),
      ],
  )


def _make_tc_layer(N, N_pad, D, residual, has_next):
  def body(*refs):
    if residual:
      (parts_ref, hprev_ref, dinv_ref, b_ref, g_ref, be_ref, xprev_ref,
       *rest) = refs
    else:
      parts_ref, hprev_ref, dinv_ref, b_ref, g_ref, be_ref, *rest = refs
    if has_next:
      wn_ref, x_out, h_out = rest
    else:
      (x_out,) = rest

    dinv = dinv_ref[...]
    agg = parts_ref[0] + parts_ref[1] + hprev_ref[...]
    conv = agg * dinv + b_ref[...]
    rid = lax.broadcasted_iota(jnp.int32, (N_pad, 1), 0)
    mask = rid < N
    r = jnp.where(mask, jax.nn.relu(conv), 0.0)
    mu = jnp.sum(r, axis=0, keepdims=True) * (1.0 / N)
    ctr = jnp.where(mask, r - mu, 0.0)
    var = jnp.sum(ctr * ctr, axis=0, keepdims=True) * (1.0 / N)
    xn = (r - mu) * lax.rsqrt(var + 1e-5) * g_ref[...] + be_ref[...]
    if residual:
      xn = xn + xprev_ref[...]
    x_out[...] = xn
    if has_next:
      h = jnp.dot(xn, wn_ref[...], preferred_element_type=jnp.float32)
      h_out[...] = h * dinv

  out_shape = [jax.ShapeDtypeStruct((N_pad, D), jnp.float32)]
  if has_next:
    out_shape.append(jax.ShapeDtypeStruct((N_pad, D), jnp.float32))
  return pl.pallas_call(body, out_shape=out_shape)


def _make_tc_final(N_pad, D):
  def body(x_ref, batch_ref, wr_ref, br_ref, pred_ref, mean_ref):
    bvals = batch_ref[...]
    gid = lax.broadcasted_iota(jnp.int32, (N_pad, G), 1)
    m = (bvals == gid).astype(jnp.float32)
    dn = (((0,), (0,)), ((), ()))
    sums = lax.dot_general(m, x_ref[...], dn,
                           preferred_element_type=jnp.float32)
    ones = jnp.ones((N_pad, 1), jnp.float32)
    counts = lax.dot_general(m, ones, dn,
                             preferred_element_type=jnp.float32)
    mean = sums / jnp.maximum(counts, 1.0)
    mean_ref[...] = mean
    pred_ref[...] = jnp.dot(mean, wr_ref[...],
                            preferred_element_type=jnp.float32) + br_ref[...]

  return pl.pallas_call(
      body,
      out_shape=[
          jax.ShapeDtypeStruct((G, 1), jnp.float32),
          jax.ShapeDtypeStruct((G, D), jnp.float32),
      ],
  )


def kernel(X, edge_index, batch, W1, b1, g1, be1, W2, b2, g2, be2, W3, b3, g3,
           be3, W4, b4, g4, be4, W5, b5, g5, be5, Wr, br):
  N, D = X.shape
  E = edge_index.shape[1]

  rows_unit = NS * 128
  N_pad = ((N + 1 + rows_unit - 1) // rows_unit) * rows_unit
  e_unit = NC * NS * CH
  E_pad = ((E + e_unit - 1) // e_unit) * e_unit

  Xp = jnp.pad(X, ((0, N_pad - N), (0, 0)))
  src = jnp.pad(edge_index[0], (0, E_pad - E), constant_values=N)
  dst = jnp.pad(edge_index[1], (0, E_pad - E), constant_values=N)
  batch2 = jnp.pad(batch, (0, N_pad - N), constant_values=G).reshape(N_pad, 1)

  sc_agg = _make_sc_agg(N_pad, E_pad, D)
  tc_first = _make_tc_first(N, N_pad, D)
  tc_final = _make_tc_final(N_pad, D)

  deg_parts = sc_agg(jnp.ones((N_pad, D), jnp.float32), src, dst)
  dinv, hp = tc_first(deg_parts, Xp, W1)

  layer_params = [
      (b1, g1, be1, W2),
      (b2, g2, be2, W3),
      (b3, g3, be3, W4),
      (b4, g4, be4, W5),
      (b5, g5, be5, None),
  ]
  x_cur = None
  for i, (b, g, be, wn) in enumerate(layer_params):
    parts = sc_agg(hp, src, dst)
    residual = i > 0
    has_next = wn is not None
    tc_layer = _make_tc_layer(N, N_pad, D, residual, has_next)
    args = [parts, hp, dinv, b.reshape(1, D), g.reshape(1, D),
            be.reshape(1, D)]
    if residual:
      args.append(x_cur)
    if has_next:
      args.append(wn)
      x_cur, hp = tc_layer(*args)
    else:
      (x_cur,) = tc_layer(*args)

  pred, mean = tc_final(x_cur, batch2, Wr, br.reshape(1, 1))
  return (pred, mean)


# speedup vs baseline: 8.9715x; 1.2165x over previous
import functools

import jax
import jax.numpy as jnp
from jax import lax
from jax.experimental import pallas as pl
from jax.experimental.pallas import tpu as pltpu
from jax.experimental.pallas import tpu_sc as plsc

NC = 2
NS = 16
L = 16
CH = 128
G = 64


def _zero_fill(buf, rows, width):
  zero = jnp.zeros((L,), jnp.float32)

  def body(r, _):
    for l in range(width // L):
      buf[r, pl.ds(l * L, L)] = zero
    return 0

  lax.fori_loop(0, rows, body, 0)


def _make_sc_agg(N_pad, E_pad, D):
  e_per_tile = E_pad // (NC * NS)
  n_ch = e_per_tile // CH
  rows_per_tile = N_pad // NS
  ZR = 128
  mesh = plsc.VectorSubcoreMesh(core_axis_name="c", subcore_axis_name="s")

  @functools.partial(
      pl.kernel,
      out_type=jax.ShapeDtypeStruct((NC, N_pad, D), jnp.float32),
      mesh=mesh,
      scratch_types=[
          pltpu.VMEM_SHARED((N_pad, D), jnp.float32),
          pltpu.VMEM((2, CH), jnp.int32),
          pltpu.VMEM((2, CH), jnp.int32),
          pltpu.VMEM((2, CH, D), jnp.float32),
          pltpu.SemaphoreType.DMA,
          pltpu.SemaphoreType.DMA,
          pltpu.SemaphoreType.DMA,
          pltpu.SemaphoreType.DMA,
      ],
  )
  def agg_kernel(h_hbm, src_hbm, dst_hbm, out_hbm, acc, srcb, dstb, rowb,
                 gsem0, gsem1, ssem0, ssem1):
    c = lax.axis_index("c")
    s = lax.axis_index("s")

    _zero_fill(rowb.at[0], ZR, D)
    row0 = s * rows_per_tile
    for k in range(rows_per_tile // ZR):
      pltpu.sync_copy(rowb.at[0], acc.at[pl.ds(row0 + k * ZR, ZR)])
    plsc.subcore_barrier()

    base0 = (c * NS + s) * e_per_tile

    def load_idx(j, slot):
      b = pl.multiple_of(base0 + j * CH, CH)
      pltpu.sync_copy(src_hbm.at[pl.ds(b, CH)], srcb.at[slot])
      pltpu.sync_copy(dst_hbm.at[pl.ds(b, CH)], dstb.at[slot])

    load_idx(0, 0)
    pltpu.async_copy(h_hbm.at[srcb.at[0]], rowb.at[0], gsem0)

    def sdrain(slot, sem):
      pltpu.make_async_copy(h_hbm.at[srcb.at[slot]], rowb.at[slot],
                            sem).wait()

    def body(j, _):
      slot = lax.rem(j, 2)

      @pl.when(slot == 0)
      def _():
        pltpu.make_async_copy(h_hbm.at[srcb.at[0]], rowb.at[0], gsem0).wait()

        @pl.when(j + 1 < n_ch)
        def _():
          @pl.when(j >= 1)
          def _():
            sdrain(1, ssem1)
          load_idx(j + 1, 1)
          pltpu.async_copy(h_hbm.at[srcb.at[1]], rowb.at[1], gsem1)
        pltpu.async_copy(rowb.at[0], acc.at[dstb.at[0]], ssem0, add=True)

      @pl.when(slot == 1)
      def _():
        pltpu.make_async_copy(h_hbm.at[srcb.at[1]], rowb.at[1], gsem1).wait()

        @pl.when(j + 1 < n_ch)
        def _():
          sdrain(0, ssem0)
          load_idx(j + 1, 0)
          pltpu.async_copy(h_hbm.at[srcb.at[0]], rowb.at[0], gsem0)
        pltpu.async_copy(rowb.at[1], acc.at[dstb.at[1]], ssem1, add=True)

      return 0

    lax.fori_loop(0, n_ch, body, 0)
    sdrain((n_ch - 2) % 2, (ssem0, ssem1)[(n_ch - 2) % 2])
    sdrain((n_ch - 1) % 2, (ssem0, ssem1)[(n_ch - 1) % 2])
    plsc.subcore_barrier()

    for k in range(rows_per_tile // ZR):
      r = row0 + k * ZR
      pltpu.sync_copy(acc.at[pl.ds(r, ZR)], out_hbm.at[c, pl.ds(r, ZR)])

  return agg_kernel


def _make_sc_degree(N_pad, E_pad, D):
  e_per_tile = E_pad // (NC * NS)
  n_ch = e_per_tile // CH
  rows_per_tile = N_pad // NS
  ZR = 128
  mesh = plsc.VectorSubcoreMesh(core_axis_name="c", subcore_axis_name="s")

  @functools.partial(
      pl.kernel,
      out_type=jax.ShapeDtypeStruct((NC, N_pad, D), jnp.float32),
      mesh=mesh,
      scratch_types=[
          pltpu.VMEM_SHARED((N_pad, D), jnp.float32),
          pltpu.VMEM((2, CH), jnp.int32),
          pltpu.VMEM((CH, D), jnp.float32),
          pltpu.SemaphoreType.DMA,
          pltpu.SemaphoreType.DMA,
      ],
  )
  def deg_kernel(dst_hbm, out_hbm, acc, dstb, ones_v, ssem0, ssem1):
    c = lax.axis_index("c")
    s = lax.axis_index("s")

    _zero_fill(ones_v, ZR, D)
    row0 = s * rows_per_tile
    for k in range(rows_per_tile // ZR):
      pltpu.sync_copy(ones_v, acc.at[pl.ds(row0 + k * ZR, ZR)])

    one = jnp.full((L,), 1.0, jnp.float32)

    def fill_ones(r, _):
      for l in range(D // L):
        ones_v[r, pl.ds(l * L, L)] = one
      return 0

    lax.fori_loop(0, CH, fill_ones, 0)
    plsc.subcore_barrier()

    base0 = (c * NS + s) * e_per_tile

    def drain(sem):
      pltpu.make_async_copy(out_hbm.at[c, pl.ds(row0, CH)], ones_v,
                            sem).wait()

    def body(j, _):
      def step(slot, sem):
        @pl.when(j >= 2)
        def _():
          drain(sem)
        b = pl.multiple_of(base0 + j * CH, CH)
        pltpu.sync_copy(dst_hbm.at[pl.ds(b, CH)], dstb.at[slot])
        pltpu.async_copy(ones_v, acc.at[dstb.at[slot]], sem, add=True)

      @pl.when(lax.rem(j, 2) == 0)
      def _():
        step(0, ssem0)

      @pl.when(lax.rem(j, 2) == 1)
      def _():
        step(1, ssem1)

      return 0

    lax.fori_loop(0, n_ch, body, 0)
    drain((ssem0, ssem1)[(n_ch - 2) % 2])
    drain((ssem0, ssem1)[(n_ch - 1) % 2])
    plsc.subcore_barrier()

    for k in range(rows_per_tile // ZR):
      r = row0 + k * ZR
      pltpu.sync_copy(acc.at[pl.ds(r, ZR)], out_hbm.at[c, pl.ds(r, ZR)])

  return deg_kernel


def _make_tc_first(N, N_pad, D):
  def body(deg_ref, x_ref, w_ref, dinv_ref, h_ref):
    deg = deg_ref[0, :, 0:1] + deg_ref[1, :, 0:1] + 1.0
    rid = lax.broadcasted_iota(jnp.int32, (N_pad, 1), 0)
    dinv = jnp.where(rid < N, lax.rsqrt(deg), 0.0)
    dinv_ref[...] = dinv
    h = jnp.dot(x_ref[...], w_ref[...], preferred_element_type=jnp.float32)
    h_ref[...] = h * dinv

  return pl.pallas_call(
      body,
      out_shape=[
          jax.ShapeDtypeStruct((N_pad, 1), jnp.float32),
          jax.ShapeDtypeStruct((N_pad, D), jnp.float32),
      ],
  )


def _make_tc_layer(N, N_pad, D, residual, has_next):
  def body(*refs):
    if residual:
      (parts_ref, hprev_ref, dinv_ref, b_ref, g_ref, be_ref, xprev_ref,
       *rest) = refs
    else:
      parts_ref, hprev_ref, dinv_ref, b_ref, g_ref, be_ref, *rest = refs
    if has_next:
      wn_ref, x_out, h_out = rest
    else:
      (x_out,) = rest

    dinv = dinv_ref[...]
    agg = parts_ref[0] + parts_ref[1] + hprev_ref[...]
    conv = agg * dinv + b_ref[...]
    rid = lax.broadcasted_iota(jnp.int32, (N_pad, 1), 0)
    mask = rid < N
    r = jnp.where(mask, jax.nn.relu(conv), 0.0)
    mu = jnp.sum(r, axis=0, keepdims=True) * (1.0 / N)
    ctr = jnp.where(mask, r - mu, 0.0)
    var = jnp.sum(ctr * ctr, axis=0, keepdims=True) * (1.0 / N)
    xn = (r - mu) * lax.rsqrt(var + 1e-5) * g_ref[...] + be_ref[...]
    if residual:
      xn = xn + xprev_ref[...]
    x_out[...] = xn
    if has_next:
      h = jnp.dot(xn, wn_ref[...], preferred_element_type=jnp.float32)
      h_out[...] = h * dinv

  out_shape = [jax.ShapeDtypeStruct((N_pad, D), jnp.float32)]
  if has_next:
    out_shape.append(jax.ShapeDtypeStruct((N_pad, D), jnp.float32))
  return pl.pallas_call(body, out_shape=out_shape)


def _make_tc_final(N_pad, D):
  def body(x_ref, batch_ref, wr_ref, br_ref, pred_ref, mean_ref):
    bvals = batch_ref[...]
    gid = lax.broadcasted_iota(jnp.int32, (N_pad, G), 1)
    m = (bvals == gid).astype(jnp.float32)
    dn = (((0,), (0,)), ((), ()))
    sums = lax.dot_general(m, x_ref[...], dn,
                           preferred_element_type=jnp.float32)
    ones = jnp.ones((N_pad, 1), jnp.float32)
    counts = lax.dot_general(m, ones, dn,
                             preferred_element_type=jnp.float32)
    mean = sums / jnp.maximum(counts, 1.0)
    mean_ref[...] = mean
    pred_ref[...] = jnp.dot(mean, wr_ref[...],
                            preferred_element_type=jnp.float32) + br_ref[...]

  return pl.pallas_call(
      body,
      out_shape=[
          jax.ShapeDtypeStruct((G, 1), jnp.float32),
          jax.ShapeDtypeStruct((G, D), jnp.float32),
      ],
  )


def kernel(X, edge_index, batch, W1, b1, g1, be1, W2, b2, g2, be2, W3, b3, g3,
           be3, W4, b4, g4, be4, W5, b5, g5, be5, Wr, br):
  N, D = X.shape
  E = edge_index.shape[1]

  rows_unit = NS * 128
  N_pad = ((N + 1 + rows_unit - 1) // rows_unit) * rows_unit
  e_unit = NC * NS * CH
  E_pad = ((E + e_unit - 1) // e_unit) * e_unit

  Xp = jnp.pad(X, ((0, N_pad - N), (0, 0)))
  src = jnp.pad(edge_index[0], (0, E_pad - E), constant_values=N)
  dst = jnp.pad(edge_index[1], (0, E_pad - E), constant_values=N)
  batch2 = jnp.pad(batch, (0, N_pad - N), constant_values=G).reshape(N_pad, 1)

  sc_agg = _make_sc_agg(N_pad, E_pad, D)
  sc_degree = _make_sc_degree(N_pad, E_pad, D)
  tc_first = _make_tc_first(N, N_pad, D)
  tc_final = _make_tc_final(N_pad, D)

  deg_parts = sc_degree(dst)
  dinv, hp = tc_first(deg_parts, Xp, W1)

  layer_params = [
      (b1, g1, be1, W2),
      (b2, g2, be2, W3),
      (b3, g3, be3, W4),
      (b4, g4, be4, W5),
      (b5, g5, be5, None),
  ]
  x_cur = None
  for i, (b, g, be, wn) in enumerate(layer_params):
    parts = sc_agg(hp, src, dst)
    residual = i > 0
    has_next = wn is not None
    tc_layer = _make_tc_layer(N, N_pad, D, residual, has_next)
    args = [parts, hp, dinv, b.reshape(1, D), g.reshape(1, D),
            be.reshape(1, D)]
    if residual:
      args.append(x_cur)
    if has_next:
      args.append(wn)
      x_cur, hp = tc_layer(*args)
    else:
      (x_cur,) = tc_layer(*args)

  pred, mean = tc_final(x_cur, batch2, Wr, br.reshape(1, 1))
  return (pred, mean)
